# baseline (speedup 1.0000x reference)
import functools

import numpy as np
import jax
import jax.numpy as jnp
from jax import lax
from jax.experimental import pallas as pl
from jax.experimental.pallas import tpu as pltpu
from jax.experimental.pallas import tpu_sc as plsc

_N = 8192
_IN = 4096
_RED = 1024
_K = 2048
_LANES = 16
_NV = _N // _LANES


_BLK = 512


def _make_score_body(cast_bf16):
  def body(x_ref, w1t_ref, b1_ref, w2t_ref, b2_ref, g_ref, s_ref):
    xv = x_ref[...]
    if cast_bf16:
      xv = xv.astype(jnp.bfloat16)
    z1 = jnp.dot(xv, w1t_ref[...], preferred_element_type=jnp.float32)
    z1 = jnp.maximum(z1 + b1_ref[...], 0.0)
    z2 = jnp.dot(z1, w2t_ref[...], preferred_element_type=jnp.float32)
    z2 = z2 + b2_ref[...]
    s = 1.0 / (1.0 + jnp.exp(-z2))
    s_ref[...] = s.reshape(xv.shape[0]) + g_ref[...]
  return body


def _scores(x, w1t, b1r, w2t, b2r, gum, blk, cast_bf16):
  n = x.shape[0]
  return pl.pallas_call(
      _make_score_body(cast_bf16),
      grid=(n // blk,),
      in_specs=[
          pl.BlockSpec((blk, _IN), lambda i: (i, 0)),
          pl.BlockSpec((_IN, _RED), lambda i: (0, 0)),
          pl.BlockSpec((1, _RED), lambda i: (0, 0)),
          pl.BlockSpec((_RED, 1), lambda i: (0, 0)),
          pl.BlockSpec((1, 1), lambda i: (0, 0)),
          pl.BlockSpec((blk,), lambda i: (i,)),
      ],
      out_specs=pl.BlockSpec((blk,), lambda i: (i,)),
      out_shape=jax.ShapeDtypeStruct((n,), jnp.float32),
  )(x, w1t, b1r, w2t, b2r, gum)


_IMIN = np.int32(-2147483648)
_IMAXP = np.int32(2147483647)
_NBINS = 256


_TILES = 16
_VPT = _N // _TILES
_VPTV = _VPT // _LANES


def _build_keys(s_v, key_v):
  def mk_body(i, _):
    u = plsc.bitcast(s_v[pl.ds(i * _LANES, _LANES)], jnp.int32)
    m = lax.shift_right_arithmetic(u, 31)
    key_v[pl.ds(i * _LANES, _LANES)] = u ^ (m & _IMAXP)
    return 0

  lax.fori_loop(0, _VPTV, mk_body, 0, unroll=8)


def _find_threshold(sid, key_v, histl_v, stage_v, merg_v, sh_v):
  zeros16 = jnp.zeros((_LANES,), jnp.int32)
  ones = jnp.ones((_LANES,), jnp.int32)
  iota16 = lax.iota(jnp.int32, _LANES)
  if True:
    prefix = jnp.int32(0)
    n_gt = jnp.int32(0)
    for lvl in range(4):
      shift = 24 - 8 * lvl

      for j in range(_LANES):
        histl_v[j] = zeros16

      def hist_body(i, _, shift=shift, prefix=prefix):
        ku = key_v[pl.ds(i * _LANES, _LANES)] ^ _IMIN
        byte = lax.shift_right_logical(ku, shift) & jnp.int32(0xFF)
        hi = lax.shift_right_logical(byte, 4)
        lo = byte & jnp.int32(0xF)
        if shift == 24:
          plsc.addupdate_scatter(histl_v, [hi, lo], ones)
        else:
          sel = lax.shift_right_logical(ku, shift + 8) == prefix
          plsc.addupdate_scatter(histl_v, [hi, lo], ones, mask=sel)
        return 0

      lax.fori_loop(0, _VPTV, hist_body, 0, unroll=4)

      for j in range(_LANES):
        stage_v[pl.ds(j * _LANES, _LANES)] = histl_v[j]
      half = (lvl % 2) * (_TILES * _NBINS)
      pltpu.sync_copy(
          stage_v, sh_v.at[pl.ds(half + sid * _NBINS, _NBINS)])
      plsc.subcore_barrier()
      pltpu.sync_copy(sh_v.at[pl.ds(half, _TILES * _NBINS)], merg_v)

      for j in range(_LANES):
        acc = merg_v[pl.ds(j * _LANES, _LANES)]
        for t in range(1, _TILES):
          acc = acc + merg_v[pl.ds(t * _NBINS + j * _LANES, _LANES)]
        histl_v[j] = acc

      need = jnp.int32(_K) - n_gt
      cum = jnp.int32(0)
      b_sel = jnp.int32(-1)
      n_above = jnp.int32(0)
      for j in range(_LANES - 1, -1, -1):
        v = histl_v[j]
        rv = lax.rev(v, (0,))
        pc = plsc.cumsum(rv)
        hit = (cum + pc) >= need
        bin_vec = jnp.int32(j * _LANES + _LANES - 1) - iota16
        b_here = jnp.max(jnp.where(hit, bin_vec, jnp.int32(-1)))
        above_here = cum + jnp.sum(jnp.where(hit, jnp.int32(0), rv))
        found_now = jnp.logical_and(b_sel < 0, b_here >= 0)
        b_sel = jnp.where(found_now, b_here, b_sel)
        n_above = jnp.where(found_now, above_here, n_above)
        cum = cum + jnp.sum(rv)

      n_gt = n_gt + n_above
      prefix = (prefix << 8) | b_sel

  thr = prefix ^ _IMIN
  return thr, n_gt


def _emit_gate(sid, thr, n_gt, key_v, gate_v, stage_v, sh_v):
  zeros16 = jnp.zeros((_LANES,), jnp.int32)
  iota16 = lax.iota(jnp.int32, _LANES)
  r_ties = jnp.int32(_K) - n_gt

  def eq_body(i, acc):
    kv = key_v[pl.ds(i * _LANES, _LANES)]
    return acc + jnp.where(kv == thr, jnp.int32(1), jnp.int32(0))

  acc = lax.fori_loop(0, _VPTV, eq_body, zeros16, unroll=4)
  my_eq = jnp.sum(acc)

  stage_v[pl.ds(0, _LANES)] = jnp.where(iota16 == 0, my_eq, jnp.int32(0))
  pltpu.sync_copy(stage_v.at[pl.ds(0, _LANES)],
                  sh_v.at[pl.ds(sid * _LANES, _LANES)])
  plsc.subcore_barrier()
  pltpu.sync_copy(sh_v.at[pl.ds(0, _TILES * _LANES)], stage_v)
  run0 = jnp.int32(0)
  for t in range(_TILES):
    cvec = stage_v[pl.ds(t * _LANES, _LANES)]
    run0 = run0 + jnp.where(jnp.int32(t) < sid, cvec[0], jnp.int32(0))

  def gate_body(i, run):
    kv = key_v[pl.ds(i * _LANES, _LANES)]
    gt = kv > thr
    eq = kv == thr
    eqi = jnp.where(eq, jnp.int32(1), jnp.int32(0))
    incl = plsc.cumsum(eqi)
    rank = run + (incl - eqi)
    sel = jnp.logical_or(gt, jnp.logical_and(eq, rank < r_ties))
    gate_v[pl.ds(i * _LANES, _LANES)] = jnp.where(sel, 1.0, 0.0).astype(
        jnp.float32)
    return run + incl[_LANES - 1]

  lax.fori_loop(0, _VPTV, gate_body, run0, unroll=4)


_BSLOT = 32
_BCAP = _TILES * _BSLOT
_MARGIN = np.float32(0.012)


def _band_sc_body(s_hbm, x_hbm, gum_hbm, bidx_hbm, bx_hbm, bgum_hbm,
                  s_v, key_v, histl_v, stage_v, merg_v, bidx_v, bgum_v,
                  gum_all_v, rows_v, sh_v, sem):
  cid = lax.axis_index("c")
  sid = lax.axis_index("s")

  @pl.when(cid == 0)
  def _():
    base = sid * _VPT
    pltpu.sync_copy(s_hbm.at[pl.ds(base, _VPT)], s_v)
    pltpu.sync_copy(gum_hbm, gum_all_v)
    _build_keys(s_v, key_v)
    thr, n_gt = _find_threshold(sid, key_v, histl_v, stage_v, merg_v, sh_v)

    m = lax.shift_right_arithmetic(thr, 31)
    t_f = plsc.bitcast(
        jnp.full((_LANES,), thr ^ (m & _IMAXP), jnp.int32), jnp.float32)
    lo = t_f - _MARGIN
    hi = t_f + _MARGIN
    iota16 = lax.iota(jnp.int32, _LANES)
    bidx_v[pl.ds(0, _LANES)] = jnp.zeros((_LANES,), jnp.int32)
    bidx_v[pl.ds(_LANES, _LANES)] = jnp.zeros((_LANES,), jnp.int32)

    def bext(k, cnt):
      sv = s_v[pl.ds(k * _LANES, _LANES)]
      msk = jnp.logical_and(sv >= lo, sv <= hi)
      mi = jnp.where(msk, jnp.int32(1), jnp.int32(0))
      pos = cnt + (plsc.cumsum(mi) - mi)
      msk2 = jnp.logical_and(msk, pos < _BSLOT)
      rowid = base + k * _LANES + iota16
      plsc.store_scatter(bidx_v, [pos], rowid, mask=msk2)
      return cnt + jnp.sum(mi)

    lax.fori_loop(0, _VPTV, bext, jnp.int32(0), unroll=4)

    for c in range(_BSLOT // _LANES):
      idx = bidx_v[pl.ds(c * _LANES, _LANES)]
      bgum_v[pl.ds(c * _LANES, _LANES)] = plsc.load_gather(gum_all_v, [idx])

    pltpu.sync_copy(bidx_v, bidx_hbm.at[pl.ds(sid * _BSLOT, _BSLOT)])
    pltpu.sync_copy(bgum_v, bgum_hbm.at[pl.ds(sid * _BSLOT, _BSLOT)])

    for c in range(_BSLOT // _LANES):
      pltpu.async_copy(
          x_hbm.at[bidx_v.at[pl.ds(c * _LANES, _LANES)]], rows_v, sem
      ).wait()
      pltpu.sync_copy(rows_v,
                      bx_hbm.at[pl.ds(sid * _BSLOT + c * _LANES, _LANES)])


def _gate2_sc_body(s_hbm, bidx_hbm, sband_hbm, gate_hbm,
                   s_v, key_v, gate_v, histl_v, stage_v, merg_v, bidxall_v,
                   sball_v, sh_v):
  cid = lax.axis_index("c")
  sid = lax.axis_index("s")

  @pl.when(cid == 0)
  def _():
    base = sid * _VPT
    pltpu.sync_copy(s_hbm.at[pl.ds(base, _VPT)], s_v)
    pltpu.sync_copy(bidx_hbm, bidxall_v)
    pltpu.sync_copy(sband_hbm, sball_v)

    def mrg(k, _):
      iv = bidxall_v[pl.ds(k * _LANES, _LANES)] - base
      vv = sball_v[pl.ds(k * _LANES, _LANES)]
      msk = jnp.logical_and(iv >= 0, iv < _VPT)
      plsc.store_scatter(s_v, [iv], vv, mask=msk)
      return 0

    lax.fori_loop(0, _BCAP // _LANES, mrg, 0, unroll=4)

    _build_keys(s_v, key_v)
    thr, n_gt = _find_threshold(sid, key_v, histl_v, stage_v, merg_v, sh_v)
    _emit_gate(sid, thr, n_gt, key_v, gate_v, stage_v, sh_v)
    pltpu.sync_copy(gate_v, gate_hbm.at[pl.ds(base, _VPT)])


@functools.cache
def _band_sc_kernel():
  return pl.kernel(
      _band_sc_body,
      out_type=(
          jax.ShapeDtypeStruct((_BCAP,), jnp.int32),
          jax.ShapeDtypeStruct((_BCAP, _IN), jnp.float32),
          jax.ShapeDtypeStruct((_BCAP,), jnp.float32),
      ),
      mesh=plsc.VectorSubcoreMesh(core_axis_name="c", subcore_axis_name="s"),
      scratch_types=[
          pltpu.VMEM((_VPT,), jnp.float32),
          pltpu.VMEM((_VPT,), jnp.int32),
          pltpu.VMEM((_LANES, _LANES), jnp.int32),
          pltpu.VMEM((_NBINS,), jnp.int32),
          pltpu.VMEM((_TILES * _NBINS,), jnp.int32),
          pltpu.VMEM((_BSLOT,), jnp.int32),
          pltpu.VMEM((_BSLOT,), jnp.float32),
          pltpu.VMEM((_N,), jnp.float32),
          pltpu.VMEM((_LANES, _IN), jnp.float32),
          pltpu.VMEM_SHARED((2 * _TILES * _NBINS,), jnp.int32),
          pltpu.SemaphoreType.DMA,
      ],
      compiler_params=pltpu.CompilerParams(
          needs_layout_passes=False, use_tc_tiling_on_sc=True),
  )


@functools.cache
def _gate2_sc_kernel():
  return pl.kernel(
      _gate2_sc_body,
      out_type=jax.ShapeDtypeStruct((_N,), jnp.float32),
      mesh=plsc.VectorSubcoreMesh(core_axis_name="c", subcore_axis_name="s"),
      scratch_types=[
          pltpu.VMEM((_VPT,), jnp.float32),
          pltpu.VMEM((_VPT,), jnp.int32),
          pltpu.VMEM((_VPT,), jnp.float32),
          pltpu.VMEM((_LANES, _LANES), jnp.int32),
          pltpu.VMEM((_NBINS,), jnp.int32),
          pltpu.VMEM((_TILES * _NBINS,), jnp.int32),
          pltpu.VMEM((_BCAP,), jnp.int32),
          pltpu.VMEM((_BCAP,), jnp.float32),
          pltpu.VMEM_SHARED((2 * _TILES * _NBINS,), jnp.int32),
      ],
      compiler_params=pltpu.CompilerParams(
          needs_layout_passes=False, use_tc_tiling_on_sc=True),
  )


def _mask_body(x_ref, g_ref, o_ref):
  g = g_ref[...].reshape(_BLK, 1)
  o_ref[...] = x_ref[...] * g


def _mask(x, gate):
  return pl.pallas_call(
      _mask_body,
      grid=(_N // _BLK,),
      in_specs=[
          pl.BlockSpec((_BLK, _IN), lambda i: (i, 0)),
          pl.BlockSpec((_BLK,), lambda i: (i,)),
      ],
      out_specs=pl.BlockSpec((_BLK, _IN), lambda i: (i, 0)),
      out_shape=jax.ShapeDtypeStruct((_N, _IN), jnp.float32),
  )(x, gate)


@jax.jit
def kernel(x, W1, b1, W2, b2, gumbels):
  w1t = W1.T
  w1tb = w1t.astype(jnp.bfloat16)
  b1r = b1.reshape(1, _RED)
  w2t = W2.T
  b2r = b2.reshape(1, 1)
  s_a = _scores(x, w1tb, b1r, w2t, b2r, gumbels, _BLK, True)
  bidx, bx, bgum = _band_sc_kernel()(s_a, x, gumbels)
  s_band = _scores(bx, w1t, b1r, w2t, b2r, bgum, _BCAP, False)
  gate = _gate2_sc_kernel()(s_a, bidx, s_band)
  return _mask(x, gate)


# speedup vs baseline: 1.5465x; 1.5465x over previous
import functools

import numpy as np
import jax
import jax.numpy as jnp
from jax import lax
from jax.experimental import pallas as pl
from jax.experimental.pallas import tpu as pltpu
from jax.experimental.pallas import tpu_sc as plsc

_N = 8192
_IN = 4096
_RED = 1024
_K = 2048
_LANES = 16
_NV = _N // _LANES


_BLK = 512


def _make_score_body(cast_bf16):
  def body(x_ref, w1t_ref, b1_ref, w2t_ref, b2_ref, g_ref, s_ref):
    xv = x_ref[...]
    if cast_bf16:
      xv = xv.astype(jnp.bfloat16)
    z1 = jnp.dot(xv, w1t_ref[...], preferred_element_type=jnp.float32)
    z1 = jnp.maximum(z1 + b1_ref[...], 0.0)
    z2 = jnp.dot(z1, w2t_ref[...], preferred_element_type=jnp.float32)
    z2 = z2 + b2_ref[...]
    s = 1.0 / (1.0 + jnp.exp(-z2))
    s_ref[...] = s.reshape(xv.shape[0]) + g_ref[...]
  return body


def _scores(x, w1t, b1r, w2t, b2r, gum, blk, cast_bf16):
  n = x.shape[0]
  return pl.pallas_call(
      _make_score_body(cast_bf16),
      grid=(n // blk,),
      in_specs=[
          pl.BlockSpec((blk, _IN), lambda i: (i, 0)),
          pl.BlockSpec((_IN, _RED), lambda i: (0, 0)),
          pl.BlockSpec((1, _RED), lambda i: (0, 0)),
          pl.BlockSpec((_RED, 1), lambda i: (0, 0)),
          pl.BlockSpec((1, 1), lambda i: (0, 0)),
          pl.BlockSpec((blk,), lambda i: (i,)),
      ],
      out_specs=pl.BlockSpec((blk,), lambda i: (i,)),
      out_shape=jax.ShapeDtypeStruct((n,), jnp.float32),
  )(x, w1t, b1r, w2t, b2r, gum)


_IMIN = np.int32(-2147483648)
_IMAXP = np.int32(2147483647)
_NBINS = 256


_TILES = 16
_VPT = _N // _TILES
_VPTV = _VPT // _LANES


def _build_keys(s_v, key_v):
  def mk_body(i, _):
    u = plsc.bitcast(s_v[pl.ds(i * _LANES, _LANES)], jnp.int32)
    m = lax.shift_right_arithmetic(u, 31)
    key_v[pl.ds(i * _LANES, _LANES)] = u ^ (m & _IMAXP)
    return 0

  lax.fori_loop(0, _VPTV, mk_body, 0, unroll=8)


def _find_threshold(sid, key_v, histl_v, stage_v, merg_v, sh_v):
  zeros16 = jnp.zeros((_LANES,), jnp.int32)
  ones = jnp.ones((_LANES,), jnp.int32)
  iota16 = lax.iota(jnp.int32, _LANES)
  if True:
    prefix = jnp.int32(0)
    n_gt = jnp.int32(0)
    for lvl in range(4):
      shift = 24 - 8 * lvl

      for j in range(_LANES):
        histl_v[j] = zeros16

      def hist_body(i, _, shift=shift, prefix=prefix):
        ku = key_v[pl.ds(i * _LANES, _LANES)] ^ _IMIN
        byte = lax.shift_right_logical(ku, shift) & jnp.int32(0xFF)
        hi = lax.shift_right_logical(byte, 4)
        lo = byte & jnp.int32(0xF)
        if shift == 24:
          plsc.addupdate_scatter(histl_v, [hi, lo], ones)
        else:
          sel = lax.shift_right_logical(ku, shift + 8) == prefix
          plsc.addupdate_scatter(histl_v, [hi, lo], ones, mask=sel)
        return 0

      lax.fori_loop(0, _VPTV, hist_body, 0, unroll=4)

      for j in range(_LANES):
        stage_v[pl.ds(j * _LANES, _LANES)] = histl_v[j]
      half = (lvl % 2) * (_TILES * _NBINS)
      pltpu.sync_copy(
          stage_v, sh_v.at[pl.ds(half + sid * _NBINS, _NBINS)])
      plsc.subcore_barrier()
      pltpu.sync_copy(sh_v.at[pl.ds(half, _TILES * _NBINS)], merg_v)

      for j in range(_LANES):
        acc = merg_v[pl.ds(j * _LANES, _LANES)]
        for t in range(1, _TILES):
          acc = acc + merg_v[pl.ds(t * _NBINS + j * _LANES, _LANES)]
        histl_v[j] = acc

      need = jnp.int32(_K) - n_gt
      cum = jnp.int32(0)
      b_sel = jnp.int32(-1)
      n_above = jnp.int32(0)
      for j in range(_LANES - 1, -1, -1):
        v = histl_v[j]
        rv = lax.rev(v, (0,))
        pc = plsc.cumsum(rv)
        hit = (cum + pc) >= need
        bin_vec = jnp.int32(j * _LANES + _LANES - 1) - iota16
        b_here = jnp.max(jnp.where(hit, bin_vec, jnp.int32(-1)))
        above_here = cum + jnp.sum(jnp.where(hit, jnp.int32(0), rv))
        found_now = jnp.logical_and(b_sel < 0, b_here >= 0)
        b_sel = jnp.where(found_now, b_here, b_sel)
        n_above = jnp.where(found_now, above_here, n_above)
        cum = cum + jnp.sum(rv)

      n_gt = n_gt + n_above
      prefix = (prefix << 8) | b_sel

  thr = prefix ^ _IMIN
  return thr, n_gt


def _emit_gate(sid, thr, n_gt, key_v, gate_v, stage_v, sh_v):
  zeros16 = jnp.zeros((_LANES,), jnp.int32)
  iota16 = lax.iota(jnp.int32, _LANES)
  r_ties = jnp.int32(_K) - n_gt

  def eq_body(i, acc):
    kv = key_v[pl.ds(i * _LANES, _LANES)]
    return acc + jnp.where(kv == thr, jnp.int32(1), jnp.int32(0))

  acc = lax.fori_loop(0, _VPTV, eq_body, zeros16, unroll=4)
  my_eq = jnp.sum(acc)

  stage_v[pl.ds(0, _LANES)] = jnp.where(iota16 == 0, my_eq, jnp.int32(0))
  pltpu.sync_copy(stage_v.at[pl.ds(0, _LANES)],
                  sh_v.at[pl.ds(sid * _LANES, _LANES)])
  plsc.subcore_barrier()
  pltpu.sync_copy(sh_v.at[pl.ds(0, _TILES * _LANES)], stage_v)
  run0 = jnp.int32(0)
  for t in range(_TILES):
    cvec = stage_v[pl.ds(t * _LANES, _LANES)]
    run0 = run0 + jnp.where(jnp.int32(t) < sid, cvec[0], jnp.int32(0))

  def gate_body(i, run):
    kv = key_v[pl.ds(i * _LANES, _LANES)]
    gt = kv > thr
    eq = kv == thr
    eqi = jnp.where(eq, jnp.int32(1), jnp.int32(0))
    incl = plsc.cumsum(eqi)
    rank = run + (incl - eqi)
    sel = jnp.logical_or(gt, jnp.logical_and(eq, rank < r_ties))
    gate_v[pl.ds(i * _LANES, _LANES)] = jnp.where(sel, 1.0, 0.0).astype(
        jnp.float32)
    return run + incl[_LANES - 1]

  lax.fori_loop(0, _VPTV, gate_body, run0, unroll=4)


_BSLOT = 32
_BCAP = _TILES * _BSLOT
_MARGIN = np.float32(0.012)


def _band_sc_body(s_hbm, x_hbm, gum_hbm, bidx_hbm, bx_hbm, bgum_hbm,
                  s_v, key_v, histl_v, stage_v, merg_v, bidx_v, bgum_v,
                  gum_all_v, rows_v, sh_v, sem):
  cid = lax.axis_index("c")
  sid = lax.axis_index("s")

  @pl.when(cid == 0)
  def _():
    base = sid * _VPT
    pltpu.sync_copy(s_hbm.at[pl.ds(base, _VPT)], s_v)
    pltpu.sync_copy(gum_hbm, gum_all_v)
    _build_keys(s_v, key_v)
    thr, n_gt = _find_threshold(sid, key_v, histl_v, stage_v, merg_v, sh_v)

    m = lax.shift_right_arithmetic(thr, 31)
    t_f = plsc.bitcast(
        jnp.full((_LANES,), thr ^ (m & _IMAXP), jnp.int32), jnp.float32)
    lo = t_f - _MARGIN
    hi = t_f + _MARGIN
    iota16 = lax.iota(jnp.int32, _LANES)
    bidx_v[pl.ds(0, _LANES)] = jnp.zeros((_LANES,), jnp.int32)
    bidx_v[pl.ds(_LANES, _LANES)] = jnp.zeros((_LANES,), jnp.int32)

    def bext(k, cnt):
      sv = s_v[pl.ds(k * _LANES, _LANES)]
      msk = jnp.logical_and(sv >= lo, sv <= hi)
      mi = jnp.where(msk, jnp.int32(1), jnp.int32(0))
      pos = cnt + (plsc.cumsum(mi) - mi)
      msk2 = jnp.logical_and(msk, pos < _BSLOT)
      rowid = base + k * _LANES + iota16
      plsc.store_scatter(bidx_v, [pos], rowid, mask=msk2)
      return cnt + jnp.sum(mi)

    lax.fori_loop(0, _VPTV, bext, jnp.int32(0), unroll=4)

    for c in range(_BSLOT // _LANES):
      idx = bidx_v[pl.ds(c * _LANES, _LANES)]
      bgum_v[pl.ds(c * _LANES, _LANES)] = plsc.load_gather(gum_all_v, [idx])

    pltpu.sync_copy(bidx_v, bidx_hbm.at[pl.ds(sid * _BSLOT, _BSLOT)])
    pltpu.sync_copy(bgum_v, bgum_hbm.at[pl.ds(sid * _BSLOT, _BSLOT)])

    for c in range(_BSLOT // _LANES):
      pltpu.async_copy(
          x_hbm.at[bidx_v.at[pl.ds(c * _LANES, _LANES)]], rows_v, sem
      ).wait()
      pltpu.sync_copy(rows_v,
                      bx_hbm.at[pl.ds(sid * _BSLOT + c * _LANES, _LANES)])


def _gate2_sc_body(s_hbm, bidx_hbm, sband_hbm, gate_hbm,
                   s_v, key_v, gate_v, histl_v, stage_v, merg_v, bidxall_v,
                   sball_v, sh_v):
  cid = lax.axis_index("c")
  sid = lax.axis_index("s")

  @pl.when(cid == 0)
  def _():
    base = sid * _VPT
    pltpu.sync_copy(s_hbm.at[pl.ds(base, _VPT)], s_v)
    pltpu.sync_copy(bidx_hbm, bidxall_v)
    pltpu.sync_copy(sband_hbm, sball_v)

    def mrg(k, _):
      iv = bidxall_v[pl.ds(k * _LANES, _LANES)] - base
      vv = sball_v[pl.ds(k * _LANES, _LANES)]
      msk = jnp.logical_and(iv >= 0, iv < _VPT)
      plsc.store_scatter(s_v, [iv], vv, mask=msk)
      return 0

    lax.fori_loop(0, _BCAP // _LANES, mrg, 0, unroll=4)

    _build_keys(s_v, key_v)
    thr, n_gt = _find_threshold(sid, key_v, histl_v, stage_v, merg_v, sh_v)
    _emit_gate(sid, thr, n_gt, key_v, gate_v, stage_v, sh_v)
    pltpu.sync_copy(gate_v, gate_hbm.at[pl.ds(base, _VPT)])


@functools.cache
def _band_sc_kernel():
  return pl.kernel(
      _band_sc_body,
      out_type=(
          jax.ShapeDtypeStruct((_BCAP,), jnp.int32),
          jax.ShapeDtypeStruct((_BCAP, _IN), jnp.float32),
          jax.ShapeDtypeStruct((_BCAP,), jnp.float32),
      ),
      mesh=plsc.VectorSubcoreMesh(core_axis_name="c", subcore_axis_name="s"),
      scratch_types=[
          pltpu.VMEM((_VPT,), jnp.float32),
          pltpu.VMEM((_VPT,), jnp.int32),
          pltpu.VMEM((_LANES, _LANES), jnp.int32),
          pltpu.VMEM((_NBINS,), jnp.int32),
          pltpu.VMEM((_TILES * _NBINS,), jnp.int32),
          pltpu.VMEM((_BSLOT,), jnp.int32),
          pltpu.VMEM((_BSLOT,), jnp.float32),
          pltpu.VMEM((_N,), jnp.float32),
          pltpu.VMEM((_LANES, _IN), jnp.float32),
          pltpu.VMEM_SHARED((2 * _TILES * _NBINS,), jnp.int32),
          pltpu.SemaphoreType.DMA,
      ],
      compiler_params=pltpu.CompilerParams(
          needs_layout_passes=False, use_tc_tiling_on_sc=True),
  )


@functools.cache
def _gate2_sc_kernel():
  return pl.kernel(
      _gate2_sc_body,
      out_type=jax.ShapeDtypeStruct((_N,), jnp.float32),
      mesh=plsc.VectorSubcoreMesh(core_axis_name="c", subcore_axis_name="s"),
      scratch_types=[
          pltpu.VMEM((_VPT,), jnp.float32),
          pltpu.VMEM((_VPT,), jnp.int32),
          pltpu.VMEM((_VPT,), jnp.float32),
          pltpu.VMEM((_LANES, _LANES), jnp.int32),
          pltpu.VMEM((_NBINS,), jnp.int32),
          pltpu.VMEM((_TILES * _NBINS,), jnp.int32),
          pltpu.VMEM((_BCAP,), jnp.int32),
          pltpu.VMEM((_BCAP,), jnp.float32),
          pltpu.VMEM_SHARED((2 * _TILES * _NBINS,), jnp.int32),
      ],
      compiler_params=pltpu.CompilerParams(
          needs_layout_passes=False, use_tc_tiling_on_sc=True),
  )


def _mask_body(x_ref, g_ref, o_ref):
  g = g_ref[...].reshape(_BLK, 1)
  o_ref[...] = x_ref[...] * g


def _mask(x, gate):
  return pl.pallas_call(
      _mask_body,
      grid=(_N // _BLK,),
      in_specs=[
          pl.BlockSpec((_BLK, _IN), lambda i: (i, 0)),
          pl.BlockSpec((_BLK,), lambda i: (i,)),
      ],
      out_specs=pl.BlockSpec((_BLK, _IN), lambda i: (i, 0)),
      out_shape=jax.ShapeDtypeStruct((_N, _IN), jnp.float32),
  )(x, gate)


@jax.jit
def kernel(x, W1, b1, W2, b2, gumbels):
  w1t = W1.T
  w1tb = w1t.astype(jnp.bfloat16)
  b1r = b1.reshape(1, _RED)
  w2t = W2.T
  b2r = b2.reshape(1, 1)
  s_a = _scores(x, w1tb, b1r, w2t, b2r, gumbels, _BLK, True)
  return _mask(x, s_a)
